# Initial kernel scaffold; baseline (speedup 1.0000x reference)
import jax
import jax.numpy as jnp
from jax import lax
from jax.experimental import pallas as pl
from jax.experimental.pallas import tpu as pltpu
from jax.experimental.pallas import tpu_sc as plsc

N = 10000
E = 320000
B = 16
EPS = 1e-5

NC = 2
NS = 16
K = 128

E_PAD = ((E + NC * NS * K - 1) // (NC * NS * K)) * (NC * NS * K)
PER_SUB_AGG = E_PAD // NS
N_AGG_IT = PER_SUB_AGG // K
PER_SUB_DEG = E_PAD // (NC * NS)
N_DEG_IT = PER_SUB_DEG // K

ROWS_PER_TILE = N // NS
INIT_CHUNK = 125
N_INIT_CH = ROWS_PER_TILE // INIT_CHUNK

ND = 11000
DEG_ROWS = 688
R = 1000
NR = N // R

_mesh = plsc.VectorSubcoreMesh(core_axis_name="c", subcore_axis_name="s")


def _deg_body(dst_hbm, out_hbm, dst_v, ones_v, stage_v, deg_sh):
    c = lax.axis_index("c")
    s = lax.axis_index("s")
    wid = c * NS + s

    e0 = jnp.where(lax.iota(jnp.int32, 16) == 0, 1.0, 0.0).astype(jnp.float32)
    z = jnp.zeros((16,), jnp.float32)

    def fill_ones(r, carry):
        ones_v[r, :] = e0
        return carry

    lax.fori_loop(0, K, fill_ones, 0)

    def fill_zero(r, carry):
        stage_v[r, :] = z
        return carry

    lax.fori_loop(0, DEG_ROWS, fill_zero, 0)

    @pl.when(s < NS - 1)
    def _():
        pltpu.sync_copy(stage_v, deg_sh.at[pl.ds(s * DEG_ROWS, DEG_ROWS)])

    @pl.when(s == NS - 1)
    def _():
        last = ND - (NS - 1) * DEG_ROWS
        pltpu.sync_copy(stage_v.at[pl.ds(0, last)],
                        deg_sh.at[pl.ds((NS - 1) * DEG_ROWS, last)])

    plsc.subcore_barrier()

    def step(j, carry):
        off = wid * PER_SUB_DEG + j * K
        pltpu.sync_copy(dst_hbm.at[pl.ds(off, K)], dst_v)
        pltpu.sync_copy(ones_v, deg_sh.at[dst_v], add=True)
        return carry

    lax.fori_loop(0, N_DEG_IT, step, 0)
    plsc.subcore_barrier()

    @pl.when(s < NS - 1)
    def _():
        pltpu.sync_copy(deg_sh.at[pl.ds(s * DEG_ROWS, DEG_ROWS)], stage_v)
        pltpu.sync_copy(stage_v,
                        out_hbm.at[pl.ds(c * ND + s * DEG_ROWS, DEG_ROWS)])

    @pl.when(s == NS - 1)
    def _():
        last = ND - (NS - 1) * DEG_ROWS
        pltpu.sync_copy(deg_sh.at[pl.ds((NS - 1) * DEG_ROWS, last)],
                        stage_v.at[pl.ds(0, last)])
        pltpu.sync_copy(stage_v.at[pl.ds(0, last)],
                        out_hbm.at[pl.ds(c * ND + (NS - 1) * DEG_ROWS, last)])


_deg_kernel = pl.kernel(
    _deg_body,
    out_type=jax.ShapeDtypeStruct((NC * ND, 16), jnp.float32),
    mesh=_mesh,
    scratch_types=[
        pltpu.VMEM((K,), jnp.int32),
        pltpu.VMEM((K, 16), jnp.float32),
        pltpu.VMEM((DEG_ROWS, 16), jnp.float32),
        pltpu.VMEM_SHARED((ND, 16), jnp.float32),
    ],
)


def _make_agg(d2):
    def body(hs_hbm, src2_hbm, dst_hbm, out_hbm, src_v, dst_v, rows_v,
             acc_sh, sem):
        c = lax.axis_index("c")
        s = lax.axis_index("s")
        base_row = s * ROWS_PER_TILE

        def init(t, carry):
            r0 = base_row + t * INIT_CHUNK
            pltpu.sync_copy(hs_hbm.at[pl.ds(c * N + r0, INIT_CHUNK)],
                            rows_v.at[pl.ds(0, INIT_CHUNK)])
            pltpu.sync_copy(rows_v.at[pl.ds(0, INIT_CHUNK)],
                            acc_sh.at[pl.ds(r0, INIT_CHUNK)])
            return carry

        lax.fori_loop(0, N_INIT_CH, init, 0)
        plsc.subcore_barrier()

        def step(j, carry):
            off = s * PER_SUB_AGG + j * K
            pltpu.sync_copy(src2_hbm.at[pl.ds(c * E_PAD + off, K)], src_v)
            pltpu.sync_copy(dst_hbm.at[pl.ds(off, K)], dst_v)
            pltpu.async_copy(hs_hbm.at[src_v], rows_v, sem).wait()
            pltpu.sync_copy(rows_v, acc_sh.at[dst_v], add=True)
            return carry

        lax.fori_loop(0, N_AGG_IT, step, 0)
        plsc.subcore_barrier()

        def out(t, carry):
            r0 = base_row + t * INIT_CHUNK
            pltpu.sync_copy(acc_sh.at[pl.ds(r0, INIT_CHUNK)],
                            rows_v.at[pl.ds(0, INIT_CHUNK)])
            pltpu.sync_copy(rows_v.at[pl.ds(0, INIT_CHUNK)],
                            out_hbm.at[pl.ds(c * N + r0, INIT_CHUNK)])
            return carry

        lax.fori_loop(0, N_INIT_CH, out, 0)

    return pl.kernel(
        body,
        out_type=jax.ShapeDtypeStruct((NC * N, d2), jnp.float32),
        mesh=_mesh,
        scratch_types=[
            pltpu.VMEM((K,), jnp.int32),
            pltpu.VMEM((K,), jnp.int32),
            pltpu.VMEM((K, d2), jnp.float32),
            pltpu.VMEM_SHARED((N + 16, d2), jnp.float32),
            pltpu.SemaphoreType.DMA,
        ],
    )


_agg64 = _make_agg(64)
_agg128 = _make_agg(128)


def _dinv_body(da_ref, db_ref, out_ref):
    s = jnp.sum(da_ref[...] + db_ref[...], axis=1, keepdims=True)
    out_ref[...] = lax.rsqrt(s + 1.0)


def _dinv_tc(deg):
    return pl.pallas_call(
        _dinv_body,
        grid=(NR,),
        in_specs=[
            pl.BlockSpec((R, 16), lambda i: (i, 0)),
            pl.BlockSpec((R, 16), lambda i: (i + ND // R, 0)),
        ],
        out_specs=pl.BlockSpec((R, 1), lambda i: (i, 0)),
        out_shape=jax.ShapeDtypeStruct((N, 1), jnp.float32),
    )(deg, deg)


def _mm0_body(x_ref, w_ref, dinv_ref, out_ref):
    out_ref[...] = dinv_ref[...] * jnp.dot(
        x_ref[...], w_ref[...], preferred_element_type=jnp.float32)


def _matmul0(x, w, dinv, d2out):
    return pl.pallas_call(
        _mm0_body,
        grid=(NC, NR),
        in_specs=[
            pl.BlockSpec((R, 128), lambda o, i: (i, 0)),
            pl.BlockSpec((128, d2out), lambda o, i: (0, o)),
            pl.BlockSpec((R, 1), lambda o, i: (i, 0)),
        ],
        out_specs=pl.BlockSpec((R, d2out), lambda o, i: (o * NR + i, 0)),
        out_shape=jax.ShapeDtypeStruct((NC * N, d2out), jnp.float32),
    )(x, w, dinv)


def _mm_body(h_ref, w_ref, dinv_ref, out_ref):
    k = pl.program_id(2)
    part = jnp.dot(h_ref[...], w_ref[...], preferred_element_type=jnp.float32)

    @pl.when(k == 0)
    def _():
        out_ref[...] = part

    @pl.when(k == 1)
    def _():
        out_ref[...] = dinv_ref[...] * (out_ref[...] + part)


def _matmul(h, w, dinv, d2in, d2out):
    return pl.pallas_call(
        _mm_body,
        grid=(NC, NR, 2),
        in_specs=[
            pl.BlockSpec((R, d2in), lambda o, i, k: (k * NR + i, 0)),
            pl.BlockSpec((d2in, d2out), lambda o, i, k: (k, o)),
            pl.BlockSpec((R, 1), lambda o, i, k: (i, 0)),
        ],
        out_specs=pl.BlockSpec((R, d2out), lambda o, i, k: (o * NR + i, 0)),
        out_shape=jax.ShapeDtypeStruct((NC * N, d2out), jnp.float32),
    )(h, w, dinv)


def _stats_body(acc_ref, dinv_ref, b_ref, sum_ref, sq_ref):
    i = pl.program_id(1)
    conv = acc_ref[...] * dinv_ref[...] + b_ref[...]
    s = jnp.sum(conv, axis=0, keepdims=True)
    q = jnp.sum(conv * conv, axis=0, keepdims=True)

    @pl.when(i == 0)
    def _():
        sum_ref[...] = s
        sq_ref[...] = q

    @pl.when(i > 0)
    def _():
        sum_ref[...] += s
        sq_ref[...] += q


def _stats(acc, dinv, b2, d2):
    return pl.pallas_call(
        _stats_body,
        grid=(NC, NR),
        in_specs=[
            pl.BlockSpec((R, d2), lambda c, i: (c * NR + i, 0)),
            pl.BlockSpec((R, 1), lambda c, i: (i, 0)),
            pl.BlockSpec((1, d2), lambda c, i: (c, 0)),
        ],
        out_specs=[
            pl.BlockSpec((1, d2), lambda c, i: (c, 0)),
            pl.BlockSpec((1, d2), lambda c, i: (c, 0)),
        ],
        out_shape=[
            jax.ShapeDtypeStruct((NC, d2), jnp.float32),
            jax.ShapeDtypeStruct((NC, d2), jnp.float32),
        ],
    )(acc, dinv, b2)


def _apply_body(acc_ref, dinv_ref, b_ref, g_ref, bb_ref, sum_ref, sq_ref,
                out_ref):
    mu = sum_ref[...] / N
    var = sq_ref[...] / N - mu * mu
    scale = g_ref[...] * lax.rsqrt(var + EPS)
    shift = bb_ref[...] - mu * scale
    conv = acc_ref[...] * dinv_ref[...] + b_ref[...]
    out_ref[...] = jnp.maximum(conv * scale + shift, 0.0)


def _bn_apply(acc, dinv, b2, g2, bb2, ssum, ssq, d2):
    return pl.pallas_call(
        _apply_body,
        grid=(NC, NR),
        in_specs=[
            pl.BlockSpec((R, d2), lambda c, i: (c * NR + i, 0)),
            pl.BlockSpec((R, 1), lambda c, i: (i, 0)),
            pl.BlockSpec((1, d2), lambda c, i: (c, 0)),
            pl.BlockSpec((1, d2), lambda c, i: (c, 0)),
            pl.BlockSpec((1, d2), lambda c, i: (c, 0)),
            pl.BlockSpec((1, d2), lambda c, i: (c, 0)),
            pl.BlockSpec((1, d2), lambda c, i: (c, 0)),
        ],
        out_specs=pl.BlockSpec((R, d2), lambda c, i: (c * NR + i, 0)),
        out_shape=jax.ShapeDtypeStruct((NC * N, d2), jnp.float32),
    )(acc, dinv, b2, g2, bb2, ssum, ssq)


def _pool_body(h_ref, batch_ref, sum_ref, cnt_ref):
    c = pl.program_id(0)
    i = pl.program_id(1)
    bids = jnp.reshape(batch_ref[...], (1, R))
    onehot = (lax.broadcasted_iota(jnp.int32, (B, R), 0) == bids
              ).astype(jnp.float32)
    part = jnp.dot(onehot, h_ref[...], preferred_element_type=jnp.float32)

    @pl.when((c == 0) & (i == 0))
    def _():
        sum_ref[...] = jnp.zeros_like(sum_ref)
        cnt_ref[...] = jnp.zeros_like(cnt_ref)

    sum_ref[0] += part

    @pl.when(c == 0)
    def _():
        cnt_ref[...] += jnp.sum(onehot, axis=1, keepdims=True)


def _pool(h3, batch2d, d2):
    return pl.pallas_call(
        _pool_body,
        grid=(NC, NR),
        in_specs=[
            pl.BlockSpec((R, d2), lambda c, i: (c * NR + i, 0)),
            pl.BlockSpec((R, 1), lambda c, i: (i, 0)),
        ],
        out_specs=[
            pl.BlockSpec((1, B, d2), lambda c, i: (c, 0, 0)),
            pl.BlockSpec((B, 1), lambda c, i: (0, 0)),
        ],
        out_shape=[
            jax.ShapeDtypeStruct((NC, B, d2), jnp.float32),
            jax.ShapeDtypeStruct((B, 1), jnp.float32),
        ],
    )(h3, batch2d)


def _head_body(sum_ref, cnt_ref, w0_ref, b0_ref, w1_ref, b1_ref, out_ref):
    cnt = jnp.maximum(cnt_ref[...], 1.0)
    p0 = sum_ref[0] / cnt
    p1 = sum_ref[1] / cnt
    o1 = (jnp.dot(p0, w0_ref[0:128, :], preferred_element_type=jnp.float32)
          + jnp.dot(p1, w0_ref[128:256, :], preferred_element_type=jnp.float32)
          + jnp.reshape(b0_ref[...], (1, 1024)))
    o1 = jnp.maximum(o1, 0.0)
    out_ref[...] = (jnp.dot(o1, w1_ref[...], preferred_element_type=jnp.float32)
                    + jnp.reshape(b1_ref[...], (1, 128)))


def _head(sums, counts, fc_w0, fc_b0, fc_w1, fc_b1):
    return pl.pallas_call(
        _head_body,
        out_shape=jax.ShapeDtypeStruct((B, 128), jnp.float32),
    )(sums, counts, fc_w0, fc_b0, fc_w1, fc_b1)


def kernel(x, edge_index, batch,
           gcn_w0, gcn_b0, bn_g0, bn_b0,
           gcn_w1, gcn_b1, bn_g1, bn_b1,
           gcn_w2, gcn_b2, bn_g2, bn_b2,
           fc_w0, fc_b0, fc_w1, fc_b1):
    src = edge_index[0]
    dst = edge_index[1]

    pad = E_PAD - E
    ar = lax.iota(jnp.int32, pad)
    src_p = jnp.concatenate([src, ar % N])
    dst_deg = jnp.concatenate([dst, N + (ar % 1000)])
    dst_agg = jnp.concatenate([dst, N + (ar % 16)])
    src2 = jnp.concatenate([src_p, src_p + N])

    deg = _deg_kernel(dst_deg)
    dinv = _dinv_tc(deg)

    batch2d = jnp.reshape(batch, (N, 1))

    h = x
    layer_params = [
        (gcn_w0, gcn_b0, bn_g0, bn_b0, 128, 64),
        (gcn_w1, gcn_b1, bn_g1, bn_b1, 64, 128),
        (gcn_w2, gcn_b2, bn_g2, bn_b2, 128, 128),
    ]
    for li, (w, b, g, bb, d2in, d2out) in enumerate(layer_params):
        if li == 0:
            hs = _matmul0(h, w, dinv, d2out)
        else:
            hs = _matmul(h, w, dinv, d2in, d2out)
        agg = _agg64 if d2out == 64 else _agg128
        acc = agg(hs, src2, dst_agg)
        b2 = jnp.reshape(b, (NC, d2out))
        g2 = jnp.reshape(g, (NC, d2out))
        bb2 = jnp.reshape(bb, (NC, d2out))
        ssum, ssq = _stats(acc, dinv, b2, d2out)
        h = _bn_apply(acc, dinv, b2, g2, bb2, ssum, ssq, d2out)

    sums, counts = _pool(h, batch2d, 128)
    return _head(sums, counts, fc_w0, fc_b0, fc_w1, fc_b1)


# speedup vs baseline: 8.6834x; 8.6834x over previous
import jax
import jax.numpy as jnp
from jax import lax
from jax.experimental import pallas as pl
from jax.experimental.pallas import tpu as pltpu
from jax.experimental.pallas import tpu_sc as plsc

N = 10000
E = 320000
B = 16
EPS = 1e-5

NC = 2
NS = 16
K = 128

E_PAD = ((E + NC * NS * K - 1) // (NC * NS * K)) * (NC * NS * K)
PER_SUB_AGG = E_PAD // NS
N_AGG_IT = PER_SUB_AGG // K
PER_SUB_DEG = E_PAD // (NC * NS)
N_DEG_IT = PER_SUB_DEG // K

ROWS_PER_TILE = 624
INIT_CHUNK = 208
N_INIT_CH = 3
REM_ROWS = N - NS * ROWS_PER_TILE

R = 1000
NR = N // R

_mesh = plsc.VectorSubcoreMesh(core_axis_name="c", subcore_axis_name="s")


def _make_agg(edge_split):
    D2 = 128

    def body(hs_hbm, src2_hbm, dst_hbm, out_hbm, src_v, dst_v, rows_v,
             stage_v, acc_sh, sem):
        c = lax.axis_index("c")
        s = lax.axis_index("s")
        base_row = s * ROWS_PER_TILE

        if edge_split:
            z = jnp.zeros((16,), jnp.float32)

            def fill_zero(r, carry):
                for j in range(D2 // 16):
                    stage_v[r, pl.ds(16 * j, 16)] = z
                return carry

            @pl.when(c == 1)
            def _():
                lax.fori_loop(0, INIT_CHUNK, fill_zero, 0)

        seed_from_hs = (c == 0) if edge_split else (s == s)
        hs_base = 0 if edge_split else c * N

        def init(t, carry):
            r0 = base_row + t * INIT_CHUNK

            @pl.when(seed_from_hs)
            def _():
                pltpu.sync_copy(hs_hbm.at[pl.ds(hs_base + r0, INIT_CHUNK)],
                                stage_v)

            pltpu.sync_copy(stage_v, acc_sh.at[pl.ds(r0, INIT_CHUNK)])
            return carry

        lax.fori_loop(0, N_INIT_CH, init, 0)

        @pl.when(s == 0)
        def _():
            r0 = NS * ROWS_PER_TILE

            @pl.when(seed_from_hs)
            def _():
                pltpu.sync_copy(hs_hbm.at[pl.ds(hs_base + r0, REM_ROWS)],
                                stage_v.at[pl.ds(0, REM_ROWS)])

            pltpu.sync_copy(stage_v.at[pl.ds(0, REM_ROWS)],
                            acc_sh.at[pl.ds(r0, REM_ROWS)])

        plsc.subcore_barrier()

        if edge_split:
            n_it = N_DEG_IT

            def offsets(j):
                off = (c * NS + s) * PER_SUB_DEG + j * K
                return off, off
        else:
            n_it = N_AGG_IT

            def offsets(j):
                off = s * PER_SUB_AGG + j * K
                return c * E_PAD + off, off

        def step(j, carry):
            src_off, dst_off = offsets(j)
            pltpu.sync_copy(src2_hbm.at[pl.ds(src_off, K)], src_v)
            pltpu.sync_copy(dst_hbm.at[pl.ds(dst_off, K)], dst_v)
            pltpu.async_copy(hs_hbm.at[src_v], rows_v, sem).wait()
            pltpu.sync_copy(rows_v, acc_sh.at[dst_v], add=True)
            return carry

        lax.fori_loop(0, n_it, step, 0)
        plsc.subcore_barrier()

        def out(t, carry):
            r0 = base_row + t * INIT_CHUNK
            pltpu.sync_copy(acc_sh.at[pl.ds(r0, INIT_CHUNK)], stage_v)
            pltpu.sync_copy(stage_v,
                            out_hbm.at[pl.ds(c * N + r0, INIT_CHUNK)])
            return carry

        lax.fori_loop(0, N_INIT_CH, out, 0)

        @pl.when(s == 0)
        def _():
            r0 = NS * ROWS_PER_TILE
            pltpu.sync_copy(acc_sh.at[pl.ds(r0, REM_ROWS)],
                            stage_v.at[pl.ds(0, REM_ROWS)])
            pltpu.sync_copy(stage_v.at[pl.ds(0, REM_ROWS)],
                            out_hbm.at[pl.ds(c * N + r0, REM_ROWS)])

    return pl.kernel(
        body,
        out_type=jax.ShapeDtypeStruct((NC * N, D2), jnp.float32),
        mesh=_mesh,
        scratch_types=[
            pltpu.VMEM((K,), jnp.int32),
            pltpu.VMEM((K,), jnp.int32),
            pltpu.VMEM((K, D2), jnp.float32),
            pltpu.VMEM((INIT_CHUNK, D2), jnp.float32),
            pltpu.VMEM_SHARED((N + 16, D2), jnp.float32),
            pltpu.SemaphoreType.DMA,
        ],
    )


_agg_edge = _make_agg(True)
_agg_col = _make_agg(False)


def _dinv_body(da_ref, db_ref, out_ref):
    s = da_ref[...][:, :1] + db_ref[...][:, :1]
    out_ref[...] = lax.rsqrt(s)


def _dinv_tc(deg):
    return pl.pallas_call(
        _dinv_body,
        grid=(NR,),
        in_specs=[
            pl.BlockSpec((R, 128), lambda i: (i, 0)),
            pl.BlockSpec((R, 128), lambda i: (i + NR, 0)),
        ],
        out_specs=pl.BlockSpec((R, 1), lambda i: (i, 0)),
        out_shape=jax.ShapeDtypeStruct((N, 1), jnp.float32),
    )(deg, deg)


def _mmp_body(x_ref, w_ref, dinv_ref, out_ref):
    out_ref[...] = dinv_ref[...] * jnp.dot(
        x_ref[...], w_ref[...], preferred_element_type=jnp.float32)


def _matmul_plain(x, w, dinv):
    return pl.pallas_call(
        _mmp_body,
        grid=(NR,),
        in_specs=[
            pl.BlockSpec((R, 128), lambda i: (i, 0)),
            pl.BlockSpec((128, 128), lambda i: (0, 0)),
            pl.BlockSpec((R, 1), lambda i: (i, 0)),
        ],
        out_specs=pl.BlockSpec((R, 128), lambda i: (i, 0)),
        out_shape=jax.ShapeDtypeStruct((N, 128), jnp.float32),
    )(x, w, dinv)


def _stats0_body(pa_ref, pb_ref, dinv_ref, b_ref, sum_ref, sq_ref):
    i = pl.program_id(0)
    conv = (pa_ref[...] + pb_ref[...]) * dinv_ref[...] + b_ref[0]
    s = jnp.sum(conv, axis=0, keepdims=True)
    q = jnp.sum(conv * conv, axis=0, keepdims=True)

    @pl.when(i == 0)
    def _():
        sum_ref[0] = s
        sq_ref[0] = q

    @pl.when(i > 0)
    def _():
        sum_ref[0] += s
        sq_ref[0] += q


def _stats0(acc2, dinv, b2):
    return pl.pallas_call(
        _stats0_body,
        grid=(NR,),
        in_specs=[
            pl.BlockSpec((R, 128), lambda i: (i, 0)),
            pl.BlockSpec((R, 128), lambda i: (i + NR, 0)),
            pl.BlockSpec((R, 1), lambda i: (i, 0)),
            pl.BlockSpec((1, 1, 128), lambda i: (0, 0, 0)),
        ],
        out_specs=[
            pl.BlockSpec((1, 1, 128), lambda i: (0, 0, 0)),
            pl.BlockSpec((1, 1, 128), lambda i: (0, 0, 0)),
        ],
        out_shape=[
            jax.ShapeDtypeStruct((1, 1, 128), jnp.float32),
            jax.ShapeDtypeStruct((1, 1, 128), jnp.float32),
        ],
    )(acc2, acc2, dinv, b2)


def _apply0_body(pa_ref, pb_ref, dinv_ref, b_ref, g_ref, bb_ref, sum_ref,
                 sq_ref, out_ref):
    mu = sum_ref[0] / N
    var = sq_ref[0] / N - mu * mu
    scale = g_ref[0] * lax.rsqrt(var + EPS)
    shift = bb_ref[0] - mu * scale
    conv = (pa_ref[...] + pb_ref[...]) * dinv_ref[...] + b_ref[0]
    out_ref[...] = jnp.maximum(conv * scale + shift, 0.0)


def _bn_apply0(acc2, dinv, b2, g2, bb2, ssum, ssq):
    p11 = pl.BlockSpec((1, 1, 128), lambda i: (0, 0, 0))
    return pl.pallas_call(
        _apply0_body,
        grid=(NR,),
        in_specs=[
            pl.BlockSpec((R, 128), lambda i: (i, 0)),
            pl.BlockSpec((R, 128), lambda i: (i + NR, 0)),
            pl.BlockSpec((R, 1), lambda i: (i, 0)),
            p11, p11, p11, p11, p11,
        ],
        out_specs=pl.BlockSpec((R, 128), lambda i: (i, 0)),
        out_shape=jax.ShapeDtypeStruct((N, 128), jnp.float32),
    )(acc2, acc2, dinv, b2, g2, bb2, ssum, ssq)


def _mm0_body(x_ref, w_ref, dinv_ref, out_ref):
    out_ref[...] = dinv_ref[...] * jnp.dot(
        x_ref[...], w_ref[0], preferred_element_type=jnp.float32)


def _matmul0(x, w, dinv, d2out):
    return pl.pallas_call(
        _mm0_body,
        grid=(NC, NR),
        in_specs=[
            pl.BlockSpec((R, 128), lambda o, i: (i, 0)),
            pl.BlockSpec((1, 128, d2out), lambda o, i: (o, 0, 0)),
            pl.BlockSpec((R, 1), lambda o, i: (i, 0)),
        ],
        out_specs=pl.BlockSpec((R, d2out), lambda o, i: (o * NR + i, 0)),
        out_shape=jax.ShapeDtypeStruct((NC * N, d2out), jnp.float32),
    )(x, w, dinv)


def _mm_body(h_ref, w_ref, dinv_ref, out_ref):
    k = pl.program_id(2)
    part = jnp.dot(h_ref[...], w_ref[...], preferred_element_type=jnp.float32)

    @pl.when(k == 0)
    def _():
        out_ref[...] = part

    @pl.when(k == 1)
    def _():
        out_ref[...] = dinv_ref[...] * (out_ref[...] + part)


def _matmul(h, w, dinv, d2in, d2out):
    return pl.pallas_call(
        _mm_body,
        grid=(NC, NR, 2),
        in_specs=[
            pl.BlockSpec((R, d2in), lambda o, i, k: (k * NR + i, 0)),
            pl.BlockSpec((d2in, d2out), lambda o, i, k: (k, o)),
            pl.BlockSpec((R, 1), lambda o, i, k: (i, 0)),
        ],
        out_specs=pl.BlockSpec((R, d2out), lambda o, i, k: (o * NR + i, 0)),
        out_shape=jax.ShapeDtypeStruct((NC * N, d2out), jnp.float32),
    )(h, w, dinv)


def _stats_body(acc_ref, dinv_ref, b_ref, sum_ref, sq_ref):
    i = pl.program_id(1)
    conv = acc_ref[...] * dinv_ref[...] + b_ref[0]
    s = jnp.sum(conv, axis=0, keepdims=True)
    q = jnp.sum(conv * conv, axis=0, keepdims=True)

    @pl.when(i == 0)
    def _():
        sum_ref[0] = s
        sq_ref[0] = q

    @pl.when(i > 0)
    def _():
        sum_ref[0] += s
        sq_ref[0] += q


def _stats(acc, dinv, b2, d2):
    return pl.pallas_call(
        _stats_body,
        grid=(NC, NR),
        in_specs=[
            pl.BlockSpec((R, d2), lambda c, i: (c * NR + i, 0)),
            pl.BlockSpec((R, 1), lambda c, i: (i, 0)),
            pl.BlockSpec((1, 1, d2), lambda c, i: (c, 0, 0)),
        ],
        out_specs=[
            pl.BlockSpec((1, 1, d2), lambda c, i: (c, 0, 0)),
            pl.BlockSpec((1, 1, d2), lambda c, i: (c, 0, 0)),
        ],
        out_shape=[
            jax.ShapeDtypeStruct((NC, 1, d2), jnp.float32),
            jax.ShapeDtypeStruct((NC, 1, d2), jnp.float32),
        ],
    )(acc, dinv, b2)


def _apply_body(acc_ref, dinv_ref, b_ref, g_ref, bb_ref, sum_ref, sq_ref,
                out_ref):
    mu = sum_ref[0] / N
    var = sq_ref[0] / N - mu * mu
    scale = g_ref[0] * lax.rsqrt(var + EPS)
    shift = bb_ref[0] - mu * scale
    conv = acc_ref[...] * dinv_ref[...] + b_ref[0]
    out_ref[...] = jnp.maximum(conv * scale + shift, 0.0)


def _bn_apply(acc, dinv, b2, g2, bb2, ssum, ssq, d2):
    return pl.pallas_call(
        _apply_body,
        grid=(NC, NR),
        in_specs=[
            pl.BlockSpec((R, d2), lambda c, i: (c * NR + i, 0)),
            pl.BlockSpec((R, 1), lambda c, i: (i, 0)),
            pl.BlockSpec((1, 1, d2), lambda c, i: (c, 0, 0)),
            pl.BlockSpec((1, 1, d2), lambda c, i: (c, 0, 0)),
            pl.BlockSpec((1, 1, d2), lambda c, i: (c, 0, 0)),
            pl.BlockSpec((1, 1, d2), lambda c, i: (c, 0, 0)),
            pl.BlockSpec((1, 1, d2), lambda c, i: (c, 0, 0)),
        ],
        out_specs=pl.BlockSpec((R, d2), lambda c, i: (c * NR + i, 0)),
        out_shape=jax.ShapeDtypeStruct((NC * N, d2), jnp.float32),
    )(acc, dinv, b2, g2, bb2, ssum, ssq)


def _pool_body(h_ref, batch_ref, sum_ref, cnt_ref):
    c = pl.program_id(0)
    i = pl.program_id(1)
    bids = jnp.reshape(batch_ref[...], (1, R))
    onehot = (lax.broadcasted_iota(jnp.int32, (B, R), 0) == bids
              ).astype(jnp.float32)
    part = jnp.dot(onehot, h_ref[...], preferred_element_type=jnp.float32)

    @pl.when((c == 0) & (i == 0))
    def _():
        sum_ref[...] = jnp.zeros_like(sum_ref)
        cnt_ref[...] = jnp.zeros_like(cnt_ref)

    sum_ref[0] += part

    @pl.when(c == 0)
    def _():
        cnt_ref[...] += jnp.sum(onehot, axis=1, keepdims=True)


def _pool(h3, batch2d, d2):
    return pl.pallas_call(
        _pool_body,
        grid=(NC, NR),
        in_specs=[
            pl.BlockSpec((R, d2), lambda c, i: (c * NR + i, 0)),
            pl.BlockSpec((R, 1), lambda c, i: (i, 0)),
        ],
        out_specs=[
            pl.BlockSpec((1, B, d2), lambda c, i: (c, 0, 0)),
            pl.BlockSpec((B, 1), lambda c, i: (0, 0)),
        ],
        out_shape=[
            jax.ShapeDtypeStruct((NC, B, d2), jnp.float32),
            jax.ShapeDtypeStruct((B, 1), jnp.float32),
        ],
    )(h3, batch2d)


def _head_body(sum_ref, cnt_ref, w0_ref, b0_ref, w1_ref, b1_ref, out_ref):
    cnt = jnp.maximum(cnt_ref[...], 1.0)
    p0 = sum_ref[0] / cnt
    p1 = sum_ref[1] / cnt
    o1 = (jnp.dot(p0, w0_ref[0:128, :], preferred_element_type=jnp.float32)
          + jnp.dot(p1, w0_ref[128:256, :], preferred_element_type=jnp.float32)
          + jnp.reshape(b0_ref[...], (1, 1024)))
    o1 = jnp.maximum(o1, 0.0)
    out_ref[...] = (jnp.dot(o1, w1_ref[...], preferred_element_type=jnp.float32)
                    + jnp.reshape(b1_ref[...], (1, 128)))


def _head(sums, counts, fc_w0, fc_b0, fc_w1, fc_b1):
    return pl.pallas_call(
        _head_body,
        out_shape=jax.ShapeDtypeStruct((B, 128), jnp.float32),
    )(sums, counts, fc_w0, fc_b0, fc_w1, fc_b1)


_USE_SC_DEG = True
_USE_SC_AGG_EDGE = True
_USE_SC_AGG_COL = True


def _xla_deg(dst):
    d = jax.ops.segment_sum(jnp.ones((E,), jnp.float32), dst, num_segments=N)
    p0 = jnp.tile((d + 1.0)[:, None], (1, 128))
    return jnp.concatenate([p0, jnp.zeros((N, 128), jnp.float32)], axis=0)


def _xla_agg_edge(hs, src_p, dst_agg):
    g = jax.ops.segment_sum(hs[src_p[:E]], dst_agg[:E], num_segments=N)
    return jnp.concatenate([hs + g, jnp.zeros_like(hs)], axis=0)


def _xla_agg_col(hs, src2, dst_agg):
    out = []
    for c in range(NC):
        h = hs[c * N:(c + 1) * N]
        g = jax.ops.segment_sum(h[src2[c * E_PAD:c * E_PAD + E] - c * N],
                                dst_agg[:E], num_segments=N)
        out.append(h + g)
    return jnp.concatenate(out, axis=0)

def kernel(x, edge_index, batch,
           gcn_w0, gcn_b0, bn_g0, bn_b0,
           gcn_w1, gcn_b1, bn_g1, bn_b1,
           gcn_w2, gcn_b2, bn_g2, bn_b2,
           fc_w0, fc_b0, fc_w1, fc_b1):
    src = edge_index[0]
    dst = edge_index[1]

    pad = E_PAD - E
    ar = lax.iota(jnp.int32, pad)
    src_p = jnp.concatenate([src, ar % N])
    dst_agg = jnp.concatenate([dst, N + (ar % 16)])
    src2 = jnp.concatenate([src_p, src_p + N])

    ones_tab = jnp.ones((N, 128), jnp.float32)
    deg = (_agg_edge(ones_tab, src_p, dst_agg) if _USE_SC_DEG
           else _xla_deg(dst))
    dinv = _dinv_tc(deg)

    batch2d = jnp.reshape(batch, (N, 1))

    hs0 = _matmul_plain(x, gcn_w0, dinv)
    acc0 = (_agg_edge(hs0, src_p, dst_agg) if _USE_SC_AGG_EDGE
            else _xla_agg_edge(hs0, src_p, dst_agg))
    b2 = jnp.reshape(gcn_b0, (1, 1, 128))
    g2 = jnp.reshape(bn_g0, (1, 1, 128))
    bb2 = jnp.reshape(bn_b0, (1, 1, 128))
    ssum, ssq = _stats0(acc0, dinv, b2)
    h = _bn_apply0(acc0, dinv, b2, g2, bb2, ssum, ssq)

    for li, (w, b, g, bb) in enumerate([
            (gcn_w1, gcn_b1, bn_g1, bn_b1),
            (gcn_w2, gcn_b2, bn_g2, bn_b2)]):
        if li == 0:
            ws = jnp.transpose(jnp.reshape(w, (128, NC, 128)), (1, 0, 2))
            hs = _matmul0(h, ws, dinv, 128)
        else:
            hs = _matmul(h, w, dinv, 128, 128)
        acc = (_agg_col(hs, src2, dst_agg) if _USE_SC_AGG_COL
               else _xla_agg_col(hs, src2, dst_agg))
        b2 = jnp.reshape(b, (NC, 1, 128))
        g2 = jnp.reshape(g, (NC, 1, 128))
        bb2 = jnp.reshape(bb, (NC, 1, 128))
        ssum, ssq = _stats(acc, dinv, b2, 128)
        h = _bn_apply(acc, dinv, b2, g2, bb2, ssum, ssq, 128)

    sums, counts = _pool(h, batch2d, 128)
    return _head(sums, counts, fc_w0, fc_b0, fc_w1, fc_b1)


# speedup vs baseline: 15.6655x; 1.8041x over previous
import jax
import jax.numpy as jnp
from jax import lax
from jax.experimental import pallas as pl
from jax.experimental.pallas import tpu as pltpu
from jax.experimental.pallas import tpu_sc as plsc

N = 10000
E = 320000
B = 16
EPS = 1e-5

NC = 2
NS = 16
K = 128

E_PAD = ((E + NC * NS * K * 4 - 1) // (NC * NS * K * 4)) * (NC * NS * K * 4)
PER_SUB_AGG = E_PAD // NS
N_AGG_IT = PER_SUB_AGG // K
PER_SUB_DEG = E_PAD // (NC * NS)
N_DEG_IT = PER_SUB_DEG // K

ROWS_PER_TILE = 624
INIT_CHUNK = 104
N_INIT_CH = 6
REM_ROWS = N - NS * ROWS_PER_TILE

R = 1000
NR = N // R

_mesh = plsc.VectorSubcoreMesh(core_axis_name="c", subcore_axis_name="s")


def _make_agg(edge_split, const_ones=False):
    D2 = 128
    n_it = N_DEG_IT if (edge_split or const_ones) else N_AGG_IT

    def body(*refs):
        if const_ones:
            (dst_hbm, out_hbm, dst_big, ones_v, stage_v, acc_sh,
             si0, si1, si2, si3) = refs
        else:
            (hs_hbm, src2_hbm, dst_hbm, out_hbm, src_big, dst_big, rows_v,
             stage_v, acc_sh, si0, si1, si2, si3, sg0, sg1) = refs
            sem_g = [sg0, sg1]
        sem_i = [si0, si1, si2, si3]

        c = lax.axis_index("c")
        s = lax.axis_index("s")
        base_row = s * ROWS_PER_TILE

        if const_ones:
            def fill_one(r, carry):
                for jj in range(D2 // 16):
                    stage_v[r, pl.ds(16 * jj, 16)] = jnp.ones(
                        (16,), jnp.float32)
                return carry

            def fill_zero0(r, carry):
                for jj in range(D2 // 16):
                    stage_v[r, pl.ds(16 * jj, 16)] = jnp.zeros(
                        (16,), jnp.float32)
                return carry

            @pl.when(c == 0)
            def _():
                lax.fori_loop(0, INIT_CHUNK, fill_one, 0)

            @pl.when(c == 1)
            def _():
                lax.fori_loop(0, INIT_CHUNK, fill_zero0, 0)

            def ofill(r, carry):
                for jj in range(D2 // 16):
                    ones_v[r, pl.ds(16 * jj, 16)] = jnp.ones(
                        (16,), jnp.float32)
                return carry

            lax.fori_loop(0, K, ofill, 0)
            seed_from_hs = None
        else:
            if edge_split:
                z = jnp.zeros((16,), jnp.float32)

                def fill_zero(r, carry):
                    for jj in range(D2 // 16):
                        stage_v[r, pl.ds(16 * jj, 16)] = z
                    return carry

                @pl.when(c == 1)
                def _():
                    lax.fori_loop(0, INIT_CHUNK, fill_zero, 0)

            seed_from_hs = (c == 0) if edge_split else (s == s)
        hs_base = 0 if edge_split else c * N

        def init(t, carry):
            r0 = base_row + t * INIT_CHUNK
            if seed_from_hs is not None:
                @pl.when(seed_from_hs)
                def _():
                    pltpu.sync_copy(
                        hs_hbm.at[pl.ds(hs_base + r0, INIT_CHUNK)], stage_v)
            pltpu.sync_copy(stage_v, acc_sh.at[pl.ds(r0, INIT_CHUNK)])
            return carry

        lax.fori_loop(0, N_INIT_CH, init, 0)

        @pl.when(s == 0)
        def _():
            r0 = NS * ROWS_PER_TILE
            if seed_from_hs is not None:
                @pl.when(seed_from_hs)
                def _():
                    pltpu.sync_copy(hs_hbm.at[pl.ds(hs_base + r0, REM_ROWS)],
                                    stage_v.at[pl.ds(0, REM_ROWS)])
            pltpu.sync_copy(stage_v.at[pl.ds(0, REM_ROWS)],
                            acc_sh.at[pl.ds(r0, REM_ROWS)])

        plsc.subcore_barrier()

        if edge_split or const_ones:
            def offsets(j):
                off = (c * NS + s) * PER_SUB_DEG + j * K
                return off, off
        else:
            def offsets(j):
                off = s * PER_SUB_AGG + j * K
                return c * E_PAD + off, off

        def issue_idx(jj, slot):
            so, do = offsets(jnp.minimum(jj, n_it - 1))
            if not const_ones:
                pltpu.async_copy(src2_hbm.at[pl.ds(so, K)],
                                 src_big.at[slot], sem_i[slot])
            pltpu.async_copy(dst_hbm.at[pl.ds(do, K)],
                             dst_big.at[slot], sem_i[slot])

        def wait_idx(slot):
            if not const_ones:
                pltpu.make_async_copy(src2_hbm.at[pl.ds(0, K)],
                                      src_big.at[slot], sem_i[slot]).wait()
            pltpu.make_async_copy(dst_hbm.at[pl.ds(0, K)],
                                  dst_big.at[slot], sem_i[slot]).wait()

        if const_ones:
            issue_idx(0, 0)
            issue_idx(1, 1)
            issue_idx(2, 2)

            def quad(j4, carry):
                jb = j4 * 4
                for j0 in range(4):
                    j = jb + j0
                    s_cur = j0
                    s_pre = (j0 + 3) % 4
                    wait_idx(s_cur)
                    pltpu.sync_copy(ones_v, acc_sh.at[dst_big.at[s_cur]],
                                    add=True)
                    issue_idx(j + 3, s_pre)
                return carry

            lax.fori_loop(0, n_it // 4, quad, 0)
            wait_idx(0)
            wait_idx(1)
            wait_idx(2)
        else:
            def issue_gather(slot, rb):
                pltpu.async_copy(hs_hbm.at[src_big.at[slot]],
                                 rows_v.at[rb], sem_g[rb])

            def wait_gather(rb):
                pltpu.make_async_copy(hs_hbm.at[pl.ds(0, K)],
                                      rows_v.at[rb], sem_g[rb]).wait()

            issue_idx(0, 0)
            issue_idx(1, 1)
            issue_idx(2, 2)
            wait_idx(0)
            issue_gather(0, 0)

            def quad(j4, carry):
                jb = j4 * 4
                for j0 in range(4):
                    j = jb + j0
                    b = j0 % 2
                    nb = 1 - b
                    s_cur = j0
                    s_next = (j0 + 1) % 4
                    s_pre = (j0 + 3) % 4
                    wait_gather(b)
                    wait_idx(s_next)
                    issue_gather(s_next, nb)
                    pltpu.sync_copy(rows_v.at[b],
                                    acc_sh.at[dst_big.at[s_cur]], add=True)
                    issue_idx(j + 3, s_pre)
                return carry

            lax.fori_loop(0, n_it // 4, quad, 0)
            wait_gather(0)
            wait_idx(1)
            wait_idx(2)

        plsc.subcore_barrier()

        def out(t, carry):
            r0 = base_row + t * INIT_CHUNK
            pltpu.sync_copy(acc_sh.at[pl.ds(r0, INIT_CHUNK)], stage_v)
            pltpu.sync_copy(stage_v,
                            out_hbm.at[pl.ds(c * N + r0, INIT_CHUNK)])
            return carry

        lax.fori_loop(0, N_INIT_CH, out, 0)

        @pl.when(s == 0)
        def _():
            r0 = NS * ROWS_PER_TILE
            pltpu.sync_copy(acc_sh.at[pl.ds(r0, REM_ROWS)],
                            stage_v.at[pl.ds(0, REM_ROWS)])
            pltpu.sync_copy(stage_v.at[pl.ds(0, REM_ROWS)],
                            out_hbm.at[pl.ds(c * N + r0, REM_ROWS)])

    if const_ones:
        scratch = [
            pltpu.VMEM((4, K), jnp.int32),
            pltpu.VMEM((K, D2), jnp.float32),
            pltpu.VMEM((INIT_CHUNK, D2), jnp.float32),
            pltpu.VMEM_SHARED((N + 16, D2), jnp.float32),
            pltpu.SemaphoreType.DMA,
            pltpu.SemaphoreType.DMA,
            pltpu.SemaphoreType.DMA,
            pltpu.SemaphoreType.DMA,
        ]
    else:
        scratch = [
            pltpu.VMEM((4, K), jnp.int32),
            pltpu.VMEM((4, K), jnp.int32),
            pltpu.VMEM((2, K, D2), jnp.float32),
            pltpu.VMEM((INIT_CHUNK, D2), jnp.float32),
            pltpu.VMEM_SHARED((N + 16, D2), jnp.float32),
            pltpu.SemaphoreType.DMA,
            pltpu.SemaphoreType.DMA,
            pltpu.SemaphoreType.DMA,
            pltpu.SemaphoreType.DMA,
            pltpu.SemaphoreType.DMA,
            pltpu.SemaphoreType.DMA,
        ]

    return pl.kernel(
        body,
        out_type=jax.ShapeDtypeStruct((NC * N, D2), jnp.float32),
        mesh=_mesh,
        scratch_types=scratch,
    )


_agg_edge = _make_agg(True)
_agg_col = _make_agg(False)
_deg_agg = _make_agg(True, const_ones=True)


def _dinv_body(da_ref, db_ref, out_ref):
    s = da_ref[...][:, :1] + db_ref[...][:, :1]
    out_ref[...] = lax.rsqrt(s)


def _dinv_tc(deg):
    return pl.pallas_call(
        _dinv_body,
        grid=(NR,),
        in_specs=[
            pl.BlockSpec((R, 128), lambda i: (i, 0)),
            pl.BlockSpec((R, 128), lambda i: (i + NR, 0)),
        ],
        out_specs=pl.BlockSpec((R, 1), lambda i: (i, 0)),
        out_shape=jax.ShapeDtypeStruct((N, 1), jnp.float32),
    )(deg, deg)


def _mmp_body(x_ref, w_ref, dinv_ref, out_ref):
    out_ref[...] = dinv_ref[...] * jnp.dot(
        x_ref[...], w_ref[...], preferred_element_type=jnp.float32)


def _matmul_plain(x, w, dinv):
    return pl.pallas_call(
        _mmp_body,
        grid=(NR,),
        in_specs=[
            pl.BlockSpec((R, 128), lambda i: (i, 0)),
            pl.BlockSpec((128, 128), lambda i: (0, 0)),
            pl.BlockSpec((R, 1), lambda i: (i, 0)),
        ],
        out_specs=pl.BlockSpec((R, 128), lambda i: (i, 0)),
        out_shape=jax.ShapeDtypeStruct((N, 128), jnp.float32),
    )(x, w, dinv)


def _stats0_body(pa_ref, pb_ref, dinv_ref, b_ref, sum_ref, sq_ref):
    i = pl.program_id(0)
    conv = (pa_ref[...] + pb_ref[...]) * dinv_ref[...] + b_ref[0]
    s = jnp.sum(conv, axis=0, keepdims=True)
    q = jnp.sum(conv * conv, axis=0, keepdims=True)

    @pl.when(i == 0)
    def _():
        sum_ref[0] = s
        sq_ref[0] = q

    @pl.when(i > 0)
    def _():
        sum_ref[0] += s
        sq_ref[0] += q


def _stats0(acc2, dinv, b2):
    return pl.pallas_call(
        _stats0_body,
        grid=(NR,),
        in_specs=[
            pl.BlockSpec((R, 128), lambda i: (i, 0)),
            pl.BlockSpec((R, 128), lambda i: (i + NR, 0)),
            pl.BlockSpec((R, 1), lambda i: (i, 0)),
            pl.BlockSpec((1, 1, 128), lambda i: (0, 0, 0)),
        ],
        out_specs=[
            pl.BlockSpec((1, 1, 128), lambda i: (0, 0, 0)),
            pl.BlockSpec((1, 1, 128), lambda i: (0, 0, 0)),
        ],
        out_shape=[
            jax.ShapeDtypeStruct((1, 1, 128), jnp.float32),
            jax.ShapeDtypeStruct((1, 1, 128), jnp.float32),
        ],
    )(acc2, acc2, dinv, b2)


def _apply0_body(pa_ref, pb_ref, dinv_ref, b_ref, g_ref, bb_ref, sum_ref,
                 sq_ref, out_ref):
    mu = sum_ref[0] / N
    var = sq_ref[0] / N - mu * mu
    scale = g_ref[0] * lax.rsqrt(var + EPS)
    shift = bb_ref[0] - mu * scale
    conv = (pa_ref[...] + pb_ref[...]) * dinv_ref[...] + b_ref[0]
    out_ref[...] = jnp.maximum(conv * scale + shift, 0.0)


def _bn_apply0(acc2, dinv, b2, g2, bb2, ssum, ssq):
    p11 = pl.BlockSpec((1, 1, 128), lambda i: (0, 0, 0))
    return pl.pallas_call(
        _apply0_body,
        grid=(NR,),
        in_specs=[
            pl.BlockSpec((R, 128), lambda i: (i, 0)),
            pl.BlockSpec((R, 128), lambda i: (i + NR, 0)),
            pl.BlockSpec((R, 1), lambda i: (i, 0)),
            p11, p11, p11, p11, p11,
        ],
        out_specs=pl.BlockSpec((R, 128), lambda i: (i, 0)),
        out_shape=jax.ShapeDtypeStruct((N, 128), jnp.float32),
    )(acc2, acc2, dinv, b2, g2, bb2, ssum, ssq)


def _mm0_body(x_ref, w_ref, dinv_ref, out_ref):
    out_ref[...] = dinv_ref[...] * jnp.dot(
        x_ref[...], w_ref[0], preferred_element_type=jnp.float32)


def _matmul0(x, w, dinv, d2out):
    return pl.pallas_call(
        _mm0_body,
        grid=(NC, NR),
        in_specs=[
            pl.BlockSpec((R, 128), lambda o, i: (i, 0)),
            pl.BlockSpec((1, 128, d2out), lambda o, i: (o, 0, 0)),
            pl.BlockSpec((R, 1), lambda o, i: (i, 0)),
        ],
        out_specs=pl.BlockSpec((R, d2out), lambda o, i: (o * NR + i, 0)),
        out_shape=jax.ShapeDtypeStruct((NC * N, d2out), jnp.float32),
    )(x, w, dinv)


def _mm_body(h_ref, w_ref, dinv_ref, out_ref):
    k = pl.program_id(2)
    part = jnp.dot(h_ref[...], w_ref[...], preferred_element_type=jnp.float32)

    @pl.when(k == 0)
    def _():
        out_ref[...] = part

    @pl.when(k == 1)
    def _():
        out_ref[...] = dinv_ref[...] * (out_ref[...] + part)


def _matmul(h, w, dinv, d2in, d2out):
    return pl.pallas_call(
        _mm_body,
        grid=(NC, NR, 2),
        in_specs=[
            pl.BlockSpec((R, d2in), lambda o, i, k: (k * NR + i, 0)),
            pl.BlockSpec((d2in, d2out), lambda o, i, k: (k, o)),
            pl.BlockSpec((R, 1), lambda o, i, k: (i, 0)),
        ],
        out_specs=pl.BlockSpec((R, d2out), lambda o, i, k: (o * NR + i, 0)),
        out_shape=jax.ShapeDtypeStruct((NC * N, d2out), jnp.float32),
    )(h, w, dinv)


def _stats_body(acc_ref, dinv_ref, b_ref, sum_ref, sq_ref):
    i = pl.program_id(1)
    conv = acc_ref[...] * dinv_ref[...] + b_ref[0]
    s = jnp.sum(conv, axis=0, keepdims=True)
    q = jnp.sum(conv * conv, axis=0, keepdims=True)

    @pl.when(i == 0)
    def _():
        sum_ref[0] = s
        sq_ref[0] = q

    @pl.when(i > 0)
    def _():
        sum_ref[0] += s
        sq_ref[0] += q


def _stats(acc, dinv, b2, d2):
    return pl.pallas_call(
        _stats_body,
        grid=(NC, NR),
        in_specs=[
            pl.BlockSpec((R, d2), lambda c, i: (c * NR + i, 0)),
            pl.BlockSpec((R, 1), lambda c, i: (i, 0)),
            pl.BlockSpec((1, 1, d2), lambda c, i: (c, 0, 0)),
        ],
        out_specs=[
            pl.BlockSpec((1, 1, d2), lambda c, i: (c, 0, 0)),
            pl.BlockSpec((1, 1, d2), lambda c, i: (c, 0, 0)),
        ],
        out_shape=[
            jax.ShapeDtypeStruct((NC, 1, d2), jnp.float32),
            jax.ShapeDtypeStruct((NC, 1, d2), jnp.float32),
        ],
    )(acc, dinv, b2)


def _apply_body(acc_ref, dinv_ref, b_ref, g_ref, bb_ref, sum_ref, sq_ref,
                out_ref):
    mu = sum_ref[0] / N
    var = sq_ref[0] / N - mu * mu
    scale = g_ref[0] * lax.rsqrt(var + EPS)
    shift = bb_ref[0] - mu * scale
    conv = acc_ref[...] * dinv_ref[...] + b_ref[0]
    out_ref[...] = jnp.maximum(conv * scale + shift, 0.0)


def _bn_apply(acc, dinv, b2, g2, bb2, ssum, ssq, d2):
    return pl.pallas_call(
        _apply_body,
        grid=(NC, NR),
        in_specs=[
            pl.BlockSpec((R, d2), lambda c, i: (c * NR + i, 0)),
            pl.BlockSpec((R, 1), lambda c, i: (i, 0)),
            pl.BlockSpec((1, 1, d2), lambda c, i: (c, 0, 0)),
            pl.BlockSpec((1, 1, d2), lambda c, i: (c, 0, 0)),
            pl.BlockSpec((1, 1, d2), lambda c, i: (c, 0, 0)),
            pl.BlockSpec((1, 1, d2), lambda c, i: (c, 0, 0)),
            pl.BlockSpec((1, 1, d2), lambda c, i: (c, 0, 0)),
        ],
        out_specs=pl.BlockSpec((R, d2), lambda c, i: (c * NR + i, 0)),
        out_shape=jax.ShapeDtypeStruct((NC * N, d2), jnp.float32),
    )(acc, dinv, b2, g2, bb2, ssum, ssq)


def _pool_body(h_ref, batch_ref, sum_ref, cnt_ref):
    c = pl.program_id(0)
    i = pl.program_id(1)
    bids = jnp.reshape(batch_ref[...], (1, R))
    onehot = (lax.broadcasted_iota(jnp.int32, (B, R), 0) == bids
              ).astype(jnp.float32)
    part = jnp.dot(onehot, h_ref[...], preferred_element_type=jnp.float32)

    @pl.when((c == 0) & (i == 0))
    def _():
        sum_ref[...] = jnp.zeros_like(sum_ref)
        cnt_ref[...] = jnp.zeros_like(cnt_ref)

    sum_ref[0] += part

    @pl.when(c == 0)
    def _():
        cnt_ref[...] += jnp.sum(onehot, axis=1, keepdims=True)


def _pool(h3, batch2d, d2):
    return pl.pallas_call(
        _pool_body,
        grid=(NC, NR),
        in_specs=[
            pl.BlockSpec((R, d2), lambda c, i: (c * NR + i, 0)),
            pl.BlockSpec((R, 1), lambda c, i: (i, 0)),
        ],
        out_specs=[
            pl.BlockSpec((1, B, d2), lambda c, i: (c, 0, 0)),
            pl.BlockSpec((B, 1), lambda c, i: (0, 0)),
        ],
        out_shape=[
            jax.ShapeDtypeStruct((NC, B, d2), jnp.float32),
            jax.ShapeDtypeStruct((B, 1), jnp.float32),
        ],
    )(h3, batch2d)


def _head_body(sum_ref, cnt_ref, w0_ref, b0_ref, w1_ref, b1_ref, out_ref):
    cnt = jnp.maximum(cnt_ref[...], 1.0)
    p0 = sum_ref[0] / cnt
    p1 = sum_ref[1] / cnt
    o1 = (jnp.dot(p0, w0_ref[0:128, :], preferred_element_type=jnp.float32)
          + jnp.dot(p1, w0_ref[128:256, :], preferred_element_type=jnp.float32)
          + jnp.reshape(b0_ref[...], (1, 1024)))
    o1 = jnp.maximum(o1, 0.0)
    out_ref[...] = (jnp.dot(o1, w1_ref[...], preferred_element_type=jnp.float32)
                    + jnp.reshape(b1_ref[...], (1, 128)))


def _head(sums, counts, fc_w0, fc_b0, fc_w1, fc_b1):
    return pl.pallas_call(
        _head_body,
        out_shape=jax.ShapeDtypeStruct((B, 128), jnp.float32),
    )(sums, counts, fc_w0, fc_b0, fc_w1, fc_b1)


_USE_SC_DEG = True
_USE_SC_AGG_EDGE = True
_USE_SC_AGG_COL = True


def _xla_deg(dst):
    d = jax.ops.segment_sum(jnp.ones((E,), jnp.float32), dst, num_segments=N)
    p0 = jnp.tile((d + 1.0)[:, None], (1, 128))
    return jnp.concatenate([p0, jnp.zeros((N, 128), jnp.float32)], axis=0)


def _xla_agg_edge(hs, src_p, dst_agg):
    g = jax.ops.segment_sum(hs[src_p[:E]], dst_agg[:E], num_segments=N)
    return jnp.concatenate([hs + g, jnp.zeros_like(hs)], axis=0)


def _xla_agg_col(hs, src2, dst_agg):
    out = []
    for c in range(NC):
        h = hs[c * N:(c + 1) * N]
        g = jax.ops.segment_sum(h[src2[c * E_PAD:c * E_PAD + E] - c * N],
                                dst_agg[:E], num_segments=N)
        out.append(h + g)
    return jnp.concatenate(out, axis=0)

def kernel(x, edge_index, batch,
           gcn_w0, gcn_b0, bn_g0, bn_b0,
           gcn_w1, gcn_b1, bn_g1, bn_b1,
           gcn_w2, gcn_b2, bn_g2, bn_b2,
           fc_w0, fc_b0, fc_w1, fc_b1):
    src = edge_index[0]
    dst = edge_index[1]

    pad = E_PAD - E
    ar = lax.iota(jnp.int32, pad)
    src_p = jnp.concatenate([src, ar % N])
    dst_agg = jnp.concatenate([dst, N + (ar % 16)])
    src2 = jnp.concatenate([src_p, src_p + N])

    deg = _deg_agg(dst_agg) if _USE_SC_DEG else _xla_deg(dst)
    dinv = _dinv_tc(deg)

    batch2d = jnp.reshape(batch, (N, 1))

    hs0 = _matmul_plain(x, gcn_w0, dinv)
    acc0 = (_agg_edge(hs0, src_p, dst_agg) if _USE_SC_AGG_EDGE
            else _xla_agg_edge(hs0, src_p, dst_agg))
    b2 = jnp.reshape(gcn_b0, (1, 1, 128))
    g2 = jnp.reshape(bn_g0, (1, 1, 128))
    bb2 = jnp.reshape(bn_b0, (1, 1, 128))
    ssum, ssq = _stats0(acc0, dinv, b2)
    h = _bn_apply0(acc0, dinv, b2, g2, bb2, ssum, ssq)

    for li, (w, b, g, bb) in enumerate([
            (gcn_w1, gcn_b1, bn_g1, bn_b1),
            (gcn_w2, gcn_b2, bn_g2, bn_b2)]):
        if li == 0:
            ws = jnp.transpose(jnp.reshape(w, (128, NC, 128)), (1, 0, 2))
            hs = _matmul0(h, ws, dinv, 128)
        else:
            hs = _matmul(h, w, dinv, 128, 128)
        acc = (_agg_col(hs, src2, dst_agg) if _USE_SC_AGG_COL
               else _xla_agg_col(hs, src2, dst_agg))
        b2 = jnp.reshape(b, (NC, 1, 128))
        g2 = jnp.reshape(g, (NC, 1, 128))
        bb2 = jnp.reshape(bb, (NC, 1, 128))
        ssum, ssq = _stats(acc, dinv, b2, 128)
        h = _bn_apply(acc, dinv, b2, g2, bb2, ssum, ssq, 128)

    sums, counts = _pool(h, batch2d, 128)
    return _head(sums, counts, fc_w0, fc_b0, fc_w1, fc_b1)
